# Initial kernel scaffold; baseline (speedup 1.0000x reference)
import jax
import jax.numpy as jnp
from jax.experimental import pallas as pl


def kernel(input_ids, embed_table):
    raise NotImplementedError("write your pallas kernel here")


# speedup vs baseline: 1.4233x; 1.4233x over previous
import functools

import jax
import jax.numpy as jnp
from jax import lax
from jax.experimental import pallas as pl
from jax.experimental.pallas import tpu as pltpu
from jax.experimental.pallas import tpu_sc as plsc

_CHUNK = 128


@functools.lru_cache(maxsize=None)
def _make_embed_kernel(V, D, B, pos_len):
    info = plsc.get_sparse_core_info()
    NC, NS, L = info.num_cores, info.num_subcores, info.num_lanes
    NW = NC * NS
    assert B % (NW * _CHUNK) == 0 and D % L == 0 and pos_len % L == 0
    b_per_w = B // NW
    n_chunks = b_per_w // _CHUNK
    mesh = plsc.VectorSubcoreMesh(core_axis_name="c", subcore_axis_name="s")

    @functools.partial(
        pl.kernel,
        mesh=mesh,
        out_type=[
            jax.ShapeDtypeStruct((B, D), jnp.float32),
            jax.ShapeDtypeStruct((B,), jnp.int32),
        ],
        scratch_types=[
            pltpu.VMEM((n_chunks, _CHUNK), jnp.int32),
            pltpu.VMEM((_CHUNK, D), jnp.float32),
            pltpu.VMEM((_CHUNK, D), jnp.float32),
            pltpu.VMEM((b_per_w,), jnp.int32),
            pltpu.SemaphoreType.DMA,
            pltpu.SemaphoreType.DMA,
        ],
    )
    def embed_kernel(table_hbm, idx_hbm, out_hbm, pos_hbm,
                     idx_v, buf0, buf1, pos_v, sem0, sem1):
        wid = lax.axis_index("s") * NC + lax.axis_index("c")
        base = wid * b_per_w
        pltpu.sync_copy(idx_hbm.at[pl.ds(wid * n_chunks, n_chunks)], idx_v)
        bufs = (buf0, buf1)
        sems = (sem0, sem1)
        copies = [None] * n_chunks
        copies[0] = pltpu.async_copy(table_hbm.at[idx_v.at[0]], bufs[0], sems[0])
        pos_base = base % pos_len
        for j in range(b_per_w // L):
            pos_v[pl.ds(j * L, L)] = lax.iota(jnp.int32, L) + (pos_base + j * L)
        pltpu.sync_copy(pos_v, pos_hbm.at[pl.ds(base, b_per_w)])
        for j in range(n_chunks):
            if j + 1 < n_chunks:
                copies[j + 1] = pltpu.async_copy(
                    table_hbm.at[idx_v.at[j + 1]], bufs[(j + 1) % 2],
                    sems[(j + 1) % 2])
            copies[j].wait()
            pltpu.sync_copy(bufs[j % 2],
                            out_hbm.at[pl.ds(base + j * _CHUNK, _CHUNK)])

    return embed_kernel


def kernel(input_ids, embed_table):
    bsz, seq_len = input_ids.shape
    V, D = embed_table.shape
    B = bsz * seq_len
    idx = input_ids.reshape(B // _CHUNK, _CHUNK).astype(jnp.int32)
    emb, pos = _make_embed_kernel(V, D, B, seq_len)(embed_table, idx)
    hidden = emb.reshape(bsz, seq_len, D)
    position_ids = pos.reshape(bsz, seq_len).astype(input_ids.dtype)
    return (hidden, position_ids)


# speedup vs baseline: 1.5175x; 1.0661x over previous
import functools

import jax
import jax.numpy as jnp
from jax import lax
from jax.experimental import pallas as pl
from jax.experimental.pallas import tpu as pltpu
from jax.experimental.pallas import tpu_sc as plsc

_CHUNK = 128
_NBUF = 4


@functools.lru_cache(maxsize=None)
def _make_embed_kernel(V, D, bsz, seq_len, idx_dtype):
    info = plsc.get_sparse_core_info()
    NC, NS, L = info.num_cores, info.num_subcores, info.num_lanes
    NW = NC * NS
    B = bsz * seq_len
    assert B % (NW * _CHUNK) == 0 and D % L == 0
    b_per_w = B // NW
    n_chunks = b_per_w // _CHUNK
    assert seq_len % b_per_w == 0
    mesh = plsc.VectorSubcoreMesh(core_axis_name="c", subcore_axis_name="s")

    @functools.partial(
        pl.kernel,
        mesh=mesh,
        out_type=[
            jax.ShapeDtypeStruct((bsz, seq_len, D), jnp.float32),
            jax.ShapeDtypeStruct((bsz, seq_len), idx_dtype),
        ],
        scratch_types=(
            [pltpu.VMEM((n_chunks, _CHUNK), jnp.int32),
             pltpu.VMEM((b_per_w,), jnp.int32)]
            + [pltpu.VMEM((_CHUNK, D), jnp.float32) for _ in range(_NBUF)]
            + [pltpu.SemaphoreType.DMA for _ in range(2 * _NBUF + 1)]
        ),
    )
    def embed_kernel(table_hbm, ids_hbm, out_hbm, pos_hbm,
                     idx_v, pos_v, *bufs_and_sems):
        bufs = bufs_and_sems[:_NBUF]
        gsems = bufs_and_sems[_NBUF:2 * _NBUF]
        wsems = bufs_and_sems[2 * _NBUF:3 * _NBUF]
        isem = bufs_and_sems[3 * _NBUF]
        wid = lax.axis_index("s") * NC + lax.axis_index("c")
        base = wid * b_per_w
        row = base // seq_len
        col = base % seq_len
        idx_copies = [
            pltpu.async_copy(ids_hbm.at[row, pl.ds(col + j * _CHUNK, _CHUNK)],
                             idx_v.at[j], isem)
            for j in range(n_chunks)
        ]
        for c in idx_copies:
            c.wait()

        def start_gather(j):
            return pltpu.async_copy(table_hbm.at[idx_v.at[j]],
                                    bufs[j % _NBUF], gsems[j % _NBUF])

        def start_writeback(j):
            return pltpu.async_copy(
                bufs[j % _NBUF],
                out_hbm.at[row, pl.ds(col + j * _CHUNK, _CHUNK)],
                wsems[j % _NBUF])

        gathers = [None] * n_chunks
        writebacks = [None] * n_chunks
        for j in range(min(_NBUF, n_chunks)):
            gathers[j] = start_gather(j)

        for j in range(b_per_w // L):
            pos_v[pl.ds(j * L, L)] = lax.iota(jnp.int32, L) + (col + j * L)
        pltpu.sync_copy(pos_v, pos_hbm.at[row, pl.ds(col, b_per_w)])

        for j in range(n_chunks):
            gathers[j].wait()
            writebacks[j] = start_writeback(j)
            nxt = j - 1 + _NBUF
            if j >= 1 and nxt < n_chunks:
                writebacks[j - 1].wait()
                gathers[nxt] = start_gather(nxt)
        for j in range(max(0, n_chunks - _NBUF), n_chunks):
            writebacks[j].wait()

    return embed_kernel


def kernel(input_ids, embed_table):
    bsz, seq_len = input_ids.shape
    V, D = embed_table.shape
    ids = input_ids.astype(jnp.int32)
    fn = _make_embed_kernel(V, D, bsz, seq_len, jnp.dtype(input_ids.dtype))
    hidden, position_ids = fn(embed_table, ids)
    return (hidden, position_ids)


# speedup vs baseline: 1.5789x; 1.0405x over previous
import functools

import jax
import jax.numpy as jnp
from jax import lax
from jax.experimental import pallas as pl
from jax.experimental.pallas import tpu as pltpu
from jax.experimental.pallas import tpu_sc as plsc

_CHUNK = 128
_NBUF = 6


@functools.lru_cache(maxsize=None)
def _make_embed_kernel(V, D, bsz, seq_len, idx_dtype):
    info = plsc.get_sparse_core_info()
    NC, NS, L = info.num_cores, info.num_subcores, info.num_lanes
    NW = NC * NS
    B = bsz * seq_len
    assert B % (NW * _CHUNK) == 0 and D % L == 0
    b_per_w = B // NW
    n_chunks = b_per_w // _CHUNK
    assert seq_len % b_per_w == 0
    mesh = plsc.VectorSubcoreMesh(core_axis_name="c", subcore_axis_name="s")

    @functools.partial(
        pl.kernel,
        mesh=mesh,
        out_type=[
            jax.ShapeDtypeStruct((bsz, seq_len, D), jnp.float32),
            jax.ShapeDtypeStruct((bsz, seq_len), idx_dtype),
        ],
        scratch_types=(
            [pltpu.VMEM((b_per_w,), jnp.int32),
             pltpu.VMEM((b_per_w,), jnp.int32)]
            + [pltpu.VMEM((_CHUNK, D), jnp.float32) for _ in range(_NBUF)]
            + [pltpu.SemaphoreType.DMA for _ in range(2 * _NBUF + 1)]
        ),
    )
    def embed_kernel(table_hbm, ids_hbm, out_hbm, pos_hbm,
                     idx_v, pos_v, *bufs_and_sems):
        bufs = bufs_and_sems[:_NBUF]
        gsems = bufs_and_sems[_NBUF:2 * _NBUF]
        wsems = bufs_and_sems[2 * _NBUF:3 * _NBUF]
        isem = bufs_and_sems[3 * _NBUF]
        wid = lax.axis_index("s") * NC + lax.axis_index("c")
        base = wid * b_per_w
        row = base // seq_len
        col = base % seq_len
        pltpu.async_copy(ids_hbm.at[row, pl.ds(col, b_per_w)], idx_v,
                         isem).wait()

        def start_gather(j):
            return pltpu.async_copy(
                table_hbm.at[idx_v.at[pl.ds(j * _CHUNK, _CHUNK)]],
                bufs[j % _NBUF], gsems[j % _NBUF])

        def start_writeback(j):
            return pltpu.async_copy(
                bufs[j % _NBUF],
                out_hbm.at[row, pl.ds(col + j * _CHUNK, _CHUNK)],
                wsems[j % _NBUF])

        gathers = [None] * n_chunks
        writebacks = [None] * n_chunks
        for j in range(min(_NBUF, n_chunks)):
            gathers[j] = start_gather(j)

        for j in range(b_per_w // L):
            pos_v[pl.ds(j * L, L)] = lax.iota(jnp.int32, L) + (col + j * L)
        pltpu.sync_copy(pos_v, pos_hbm.at[row, pl.ds(col, b_per_w)])

        for j in range(n_chunks):
            gathers[j].wait()
            writebacks[j] = start_writeback(j)
            nxt = j - 1 + _NBUF
            if j >= 1 and nxt < n_chunks:
                writebacks[j - 1].wait()
                gathers[nxt] = start_gather(nxt)
        for j in range(max(0, n_chunks - _NBUF), n_chunks):
            writebacks[j].wait()

    return embed_kernel


def kernel(input_ids, embed_table):
    bsz, seq_len = input_ids.shape
    V, D = embed_table.shape
    ids = input_ids.astype(jnp.int32)
    fn = _make_embed_kernel(V, D, bsz, seq_len, jnp.dtype(input_ids.dtype))
    hidden, position_ids = fn(embed_table, ids)
    return (hidden, position_ids)


# speedup vs baseline: 1.5794x; 1.0003x over previous
import functools

import jax
import jax.numpy as jnp
from jax import lax
from jax.experimental import pallas as pl
from jax.experimental.pallas import tpu as pltpu
from jax.experimental.pallas import tpu_sc as plsc

_CHUNK = 128
_NBUF = 7


@functools.lru_cache(maxsize=None)
def _make_embed_kernel(V, D, bsz, seq_len, idx_dtype):
    info = plsc.get_sparse_core_info()
    NC, NS, L = info.num_cores, info.num_subcores, info.num_lanes
    NW = NC * NS
    B = bsz * seq_len
    assert B % (NW * _CHUNK) == 0 and D % L == 0
    b_per_w = B // NW
    n_chunks = b_per_w // _CHUNK
    assert seq_len % b_per_w == 0
    mesh = plsc.VectorSubcoreMesh(core_axis_name="c", subcore_axis_name="s")

    @functools.partial(
        pl.kernel,
        mesh=mesh,
        out_type=[
            jax.ShapeDtypeStruct((bsz, seq_len, D), jnp.float32),
            jax.ShapeDtypeStruct((bsz, seq_len), idx_dtype),
        ],
        scratch_types=(
            [pltpu.VMEM((b_per_w,), jnp.int32),
             pltpu.VMEM((b_per_w,), jnp.int32)]
            + [pltpu.VMEM((_CHUNK, D), jnp.float32) for _ in range(_NBUF)]
            + [pltpu.SemaphoreType.DMA for _ in range(2 * _NBUF + 1)]
        ),
    )
    def embed_kernel(table_hbm, ids_hbm, out_hbm, pos_hbm,
                     idx_v, pos_v, *bufs_and_sems):
        bufs = bufs_and_sems[:_NBUF]
        gsems = bufs_and_sems[_NBUF:2 * _NBUF]
        wsems = bufs_and_sems[2 * _NBUF:3 * _NBUF]
        isem = bufs_and_sems[3 * _NBUF]
        wid = lax.axis_index("s") * NC + lax.axis_index("c")
        base = wid * b_per_w
        row = base // seq_len
        col = base % seq_len
        idx_copy = pltpu.async_copy(ids_hbm.at[row, pl.ds(col, b_per_w)],
                                    idx_v, isem)
        for j in range(b_per_w // L):
            pos_v[pl.ds(j * L, L)] = lax.iota(jnp.int32, L) + (col + j * L)
        idx_copy.wait()

        def start_gather(j):
            return pltpu.async_copy(
                table_hbm.at[idx_v.at[pl.ds(j * _CHUNK, _CHUNK)]],
                bufs[j % _NBUF], gsems[j % _NBUF])

        def start_writeback(j):
            return pltpu.async_copy(
                bufs[j % _NBUF],
                out_hbm.at[row, pl.ds(col + j * _CHUNK, _CHUNK)],
                wsems[j % _NBUF])

        gathers = [None] * n_chunks
        writebacks = [None] * n_chunks
        for j in range(min(_NBUF, n_chunks)):
            gathers[j] = start_gather(j)

        pltpu.sync_copy(pos_v, pos_hbm.at[row, pl.ds(col, b_per_w)])

        for j in range(n_chunks):
            gathers[j].wait()
            writebacks[j] = start_writeback(j)
            nxt = j - 1 + _NBUF
            if j >= 1 and nxt < n_chunks:
                writebacks[j - 1].wait()
                gathers[nxt] = start_gather(nxt)
        for j in range(max(0, n_chunks - _NBUF), n_chunks):
            writebacks[j].wait()

    return embed_kernel


def kernel(input_ids, embed_table):
    bsz, seq_len = input_ids.shape
    V, D = embed_table.shape
    ids = input_ids.astype(jnp.int32)
    fn = _make_embed_kernel(V, D, bsz, seq_len, jnp.dtype(input_ids.dtype))
    hidden, position_ids = fn(embed_table, ids)
    return (hidden, position_ids)
